# baseline (speedup 1.0000x reference)
import functools

import jax
import jax.numpy as jnp
from jax import lax
from jax.experimental import pallas as pl
from jax.experimental.pallas import tpu as pltpu
from jax.experimental.pallas import tpu_sc as plsc

T, D = 2048, 768
E, K = 8, 2
FF = 4 * D

BT = 256
G = 24
R = G * BT

NC, NS = 2, 16
NW = NC * NS


def _gate_body(h_ref, w1_ref, w2_ref, a1_ref, a2_ref):
    h = h_ref[...]
    iota = lax.broadcasted_iota(jnp.int32, (T, E), 1)
    m1 = jnp.max(h, axis=1)
    a1 = jnp.min(jnp.where(h == m1[:, None], iota, E), axis=1)
    hm = jnp.where(iota == a1[:, None], -jnp.inf, h)
    m2 = jnp.max(hm, axis=1)
    a2 = jnp.min(jnp.where(hm == m2[:, None], iota, E), axis=1)
    d = jnp.exp(m2 - m1)
    w1_ref[...] = 1.0 / (1.0 + d)
    w2_ref[...] = d / (1.0 + d)
    a1_ref[...] = a1
    a2_ref[...] = a2


def _gate(h):
    return pl.pallas_call(
        _gate_body,
        out_shape=[
            jax.ShapeDtypeStruct((T,), jnp.float32),
            jax.ShapeDtypeStruct((T,), jnp.float32),
            jax.ShapeDtypeStruct((T,), jnp.int32),
            jax.ShapeDtypeStruct((T,), jnp.int32),
        ],
    )(h)


def _route(a1, a2, w1, w2):
    idx_flat = jnp.stack([a1, a2], axis=1).reshape(-1)
    w_flat = jnp.stack([w1, w2], axis=1).reshape(-1)
    onehot = (idx_flat[:, None] == jnp.arange(E)[None, :]).astype(jnp.int32)
    cs = jnp.cumsum(onehot, axis=0)
    counts = cs[-1]
    rank = jnp.sum((cs - onehot) * onehot, axis=1)
    alig = ((counts + BT - 1) // BT) * BT
    cum = jnp.cumsum(alig)
    offs = cum - alig
    pos = offs[idx_flat] + rank
    used = cum[-1] // BT
    tok = jnp.arange(T * K, dtype=jnp.int32) // K
    st = jnp.zeros((R,), jnp.int32).at[pos].set(tok)
    sw = jnp.zeros((R,), jnp.float32).at[pos].set(w_flat)
    bidx = jnp.arange(G, dtype=jnp.int32) * BT
    be_raw = jnp.searchsorted(cum, bidx, side="right").astype(jnp.int32)
    validb = bidx < cum[-1]
    last_e = jnp.clip(
        jnp.searchsorted(cum, cum[-1] - 1, side="right"), 0, E - 1
    ).astype(jnp.int32)
    be = jnp.where(validb, jnp.clip(be_raw, 0, E - 1), last_e)
    rb = jnp.where(validb, jnp.arange(G, dtype=jnp.int32),
                   jnp.maximum(used - 1, 0).astype(jnp.int32))
    meta = jnp.concatenate([be, rb, used[None].astype(jnp.int32)])
    return st, sw, pos.astype(jnp.int32), meta


_RW = R // NW
_GC = 96


def _sc_gather(x2, st):
    mesh = plsc.VectorSubcoreMesh(core_axis_name="c", subcore_axis_name="s")

    @functools.partial(
        pl.kernel, mesh=mesh,
        out_type=jax.ShapeDtypeStruct((R, D), jnp.float32),
        scratch_types=[
            pltpu.VMEM((_RW,), jnp.int32),
            pltpu.VMEM((_GC, D), jnp.float32),
            pltpu.SemaphoreType.DMA,
        ],
    )
    def k(x_hbm, st_hbm, out_hbm, idx_v, rows_v, sem):
        wid = lax.axis_index("s") * NC + lax.axis_index("c")
        base = wid * _RW
        pltpu.sync_copy(st_hbm.at[pl.ds(base, _RW)], idx_v)
        for c in range(_RW // _GC):
            pltpu.async_copy(
                x_hbm.at[idx_v.at[pl.ds(c * _GC, _GC)]], rows_v, sem
            ).wait()
            pltpu.sync_copy(rows_v, out_hbm.at[pl.ds(base + c * _GC, _GC)])

    return k(x2, st)


def _ffn_body(m_ref, xs_ref, W1_ref, b1_ref, W2_ref, b2_ref, w_ref, out_ref):
    b = pl.program_id(0)

    @pl.when(b < m_ref[2 * G])
    def _():
        h1 = lax.dot_general(
            xs_ref[...], W1_ref[0], (((1,), (0,)), ((), ())),
            preferred_element_type=jnp.float32,
            precision=lax.Precision.HIGHEST)
        h1 = jnp.maximum(h1 + b1_ref[0, 0][None, :], 0.0)
        y = lax.dot_general(
            h1, W2_ref[0], (((1,), (0,)), ((), ())),
            preferred_element_type=jnp.float32,
            precision=lax.Precision.HIGHEST)
        y = y + b2_ref[0, 0][None, :]
        out_ref[...] = y * w_ref[...][:, None]


def _ffn(meta, xs, W1, b1, W2, b2, sw):
    grid_spec = pltpu.PrefetchScalarGridSpec(
        num_scalar_prefetch=1,
        grid=(G,),
        in_specs=[
            pl.BlockSpec((BT, D), lambda b, m: (m[G + b], 0)),
            pl.BlockSpec((1, D, FF), lambda b, m: (m[b], 0, 0)),
            pl.BlockSpec((1, 1, FF), lambda b, m: (m[b], 0, 0)),
            pl.BlockSpec((1, FF, D), lambda b, m: (m[b], 0, 0)),
            pl.BlockSpec((1, 1, D), lambda b, m: (m[b], 0, 0)),
            pl.BlockSpec((BT,), lambda b, m: (m[G + b],)),
        ],
        out_specs=pl.BlockSpec((BT, D), lambda b, m: (m[G + b], 0)),
    )
    return pl.pallas_call(
        _ffn_body,
        grid_spec=grid_spec,
        out_shape=jax.ShapeDtypeStruct((R, D), jnp.float32),
        compiler_params=pltpu.CompilerParams(
            dimension_semantics=("arbitrary",)),
    )(meta, xs, W1, b1.reshape(E, 1, FF), W2, b2.reshape(E, 1, D), sw)


_TW = T // NW
_CC = 32


def _sc_combine(hw, pos):
    mesh = plsc.VectorSubcoreMesh(core_axis_name="c", subcore_axis_name="s")

    @functools.partial(
        pl.kernel, mesh=mesh,
        out_type=jax.ShapeDtypeStruct((T, D), jnp.float32),
        scratch_types=[
            pltpu.VMEM((K * _TW,), jnp.int32),
            pltpu.VMEM((K * _CC, D), jnp.float32),
            pltpu.VMEM((_CC, D), jnp.float32),
            pltpu.SemaphoreType.DMA,
        ],
    )
    def k(hw_hbm, pos_hbm, out_hbm, idx_v, rows_v, out_v, sem):
        wid = lax.axis_index("s") * NC + lax.axis_index("c")
        tbase = wid * _TW
        pltpu.sync_copy(pos_hbm.at[pl.ds(K * tbase, K * _TW)], idx_v)
        for c in range(_TW // _CC):
            pltpu.async_copy(
                hw_hbm.at[idx_v.at[pl.ds(c * K * _CC, K * _CC)]], rows_v, sem
            ).wait()

            def body(t, carry):
                for dch in range(D // 16):
                    sl = pl.ds(dch * 16, 16)
                    out_v[t, sl] = rows_v[2 * t, sl] + rows_v[2 * t + 1, sl]
                return carry

            lax.fori_loop(0, _CC, body, 0)
            pltpu.sync_copy(out_v, out_hbm.at[pl.ds(tbase + c * _CC, _CC)])

    return k(hw, pos)


def kernel(x, Wg, bg, Wn, bn, W1, b1, W2, b2):
    x2 = x[0]
    noise = jax.random.normal(jax.random.PRNGKey(42), (1, T, E),
                              dtype=jnp.float32)
    h_logits = (x @ Wg + bg + noise * jax.nn.softplus(x @ Wn + bn))[0]
    w1, w2, a1, a2 = _gate(h_logits)
    st, sw, pos, meta = _route(a1, a2, w1, w2)
    xs = _sc_gather(x2, st)
    hw = _ffn(meta, xs, W1, b1, W2, b2, sw)
    out2 = _sc_combine(hw, pos)
    return out2[None, :, :]


# speedup vs baseline: 1.7447x; 1.7447x over previous
import functools

import jax
import jax.numpy as jnp
from jax import lax
from jax.experimental import pallas as pl
from jax.experimental.pallas import tpu as pltpu
from jax.experimental.pallas import tpu_sc as plsc

T, D = 2048, 768
E, K = 8, 2
FF = 4 * D

BT = 256
G = 24
R = G * BT

NC, NS = 2, 16
NW = NC * NS


def _gate_body(h_ref, w1_ref, w2_ref, a1_ref, a2_ref):
    h = h_ref[...]
    iota = lax.broadcasted_iota(jnp.int32, (T, E), 1)
    m1 = jnp.max(h, axis=1)
    a1 = jnp.min(jnp.where(h == m1[:, None], iota, E), axis=1)
    hm = jnp.where(iota == a1[:, None], -jnp.inf, h)
    m2 = jnp.max(hm, axis=1)
    a2 = jnp.min(jnp.where(hm == m2[:, None], iota, E), axis=1)
    d = jnp.exp(m2 - m1)
    w1_ref[...] = 1.0 / (1.0 + d)
    w2_ref[...] = d / (1.0 + d)
    a1_ref[...] = a1
    a2_ref[...] = a2


def _gate(h):
    return pl.pallas_call(
        _gate_body,
        out_shape=[
            jax.ShapeDtypeStruct((T,), jnp.float32),
            jax.ShapeDtypeStruct((T,), jnp.float32),
            jax.ShapeDtypeStruct((T,), jnp.int32),
            jax.ShapeDtypeStruct((T,), jnp.int32),
        ],
    )(h)


def _route(a1, a2, w1, w2):
    idx_flat = jnp.stack([a1, a2], axis=1).reshape(-1)
    w_flat = jnp.stack([w1, w2], axis=1).reshape(-1)
    onehot = (idx_flat[:, None] == jnp.arange(E)[None, :]).astype(jnp.int32)
    cs = jnp.cumsum(onehot, axis=0)
    counts = cs[-1]
    rank = jnp.sum((cs - onehot) * onehot, axis=1)
    alig = ((counts + BT - 1) // BT) * BT
    cum = jnp.cumsum(alig)
    offs = cum - alig
    pos = offs[idx_flat] + rank
    used = cum[-1] // BT
    tok = jnp.arange(T * K, dtype=jnp.int32) // K
    st = jnp.zeros((R,), jnp.int32).at[pos].set(tok)
    sw = jnp.zeros((R,), jnp.float32).at[pos].set(w_flat)
    bidx = jnp.arange(G, dtype=jnp.int32) * BT
    be_raw = jnp.searchsorted(cum, bidx, side="right").astype(jnp.int32)
    validb = bidx < cum[-1]
    last_e = jnp.clip(
        jnp.searchsorted(cum, cum[-1] - 1, side="right"), 0, E - 1
    ).astype(jnp.int32)
    be = jnp.where(validb, jnp.clip(be_raw, 0, E - 1), last_e)
    rb = jnp.where(validb, jnp.arange(G, dtype=jnp.int32),
                   jnp.maximum(used - 1, 0).astype(jnp.int32))
    meta = jnp.concatenate([be, rb, used[None].astype(jnp.int32)])
    return st, sw, pos.astype(jnp.int32), meta


_RW = R // NW
_GC = 96


def _sc_gather(x2, st):
    mesh = plsc.VectorSubcoreMesh(core_axis_name="c", subcore_axis_name="s")

    @functools.partial(
        pl.kernel, mesh=mesh,
        out_type=jax.ShapeDtypeStruct((R, D), jnp.float32),
        scratch_types=[
            pltpu.VMEM((_RW,), jnp.int32),
            pltpu.VMEM((_GC, D), jnp.float32),
            pltpu.SemaphoreType.DMA,
        ],
    )
    def k(x_hbm, st_hbm, out_hbm, idx_v, rows_v, sem):
        wid = lax.axis_index("s") * NC + lax.axis_index("c")
        base = wid * _RW
        pltpu.sync_copy(st_hbm.at[pl.ds(base, _RW)], idx_v)
        for c in range(_RW // _GC):
            pltpu.async_copy(
                x_hbm.at[idx_v.at[pl.ds(c * _GC, _GC)]], rows_v, sem
            ).wait()
            pltpu.sync_copy(rows_v, out_hbm.at[pl.ds(base + c * _GC, _GC)])

    return k(x2, st)


def _ffn_body(m_ref, xs_ref, W1_ref, b1_ref, W2_ref, b2_ref, w_ref, out_ref):
    b = pl.program_id(0)

    @pl.when(b < m_ref[2 * G])
    def _():
        h1 = lax.dot_general(
            xs_ref[...].astype(jnp.bfloat16), W1_ref[0].astype(jnp.bfloat16),
            (((1,), (0,)), ((), ())),
            preferred_element_type=jnp.float32)
        h1 = jnp.maximum(h1 + b1_ref[0, 0][None, :], 0.0)
        y = lax.dot_general(
            h1.astype(jnp.bfloat16), W2_ref[0].astype(jnp.bfloat16),
            (((1,), (0,)), ((), ())),
            preferred_element_type=jnp.float32)
        y = y + b2_ref[0, 0][None, :]
        out_ref[...] = y * w_ref[...][:, None]


def _ffn(meta, xs, W1, b1, W2, b2, sw):
    grid_spec = pltpu.PrefetchScalarGridSpec(
        num_scalar_prefetch=1,
        grid=(G,),
        in_specs=[
            pl.BlockSpec((BT, D), lambda b, m: (m[G + b], 0)),
            pl.BlockSpec((1, D, FF), lambda b, m: (m[b], 0, 0)),
            pl.BlockSpec((1, 1, FF), lambda b, m: (m[b], 0, 0)),
            pl.BlockSpec((1, FF, D), lambda b, m: (m[b], 0, 0)),
            pl.BlockSpec((1, 1, D), lambda b, m: (m[b], 0, 0)),
            pl.BlockSpec((BT,), lambda b, m: (m[G + b],)),
        ],
        out_specs=pl.BlockSpec((BT, D), lambda b, m: (m[G + b], 0)),
    )
    return pl.pallas_call(
        _ffn_body,
        grid_spec=grid_spec,
        out_shape=jax.ShapeDtypeStruct((R, D), jnp.float32),
        compiler_params=pltpu.CompilerParams(
            dimension_semantics=("arbitrary",)),
    )(meta, xs, W1, b1.reshape(E, 1, FF), W2, b2.reshape(E, 1, D), sw)


_TW = T // NW
_CC = 32


def _sc_combine(hw, pos):
    mesh = plsc.VectorSubcoreMesh(core_axis_name="c", subcore_axis_name="s")

    @functools.partial(
        pl.kernel, mesh=mesh,
        out_type=jax.ShapeDtypeStruct((T, D), jnp.float32),
        scratch_types=[
            pltpu.VMEM((K * _TW,), jnp.int32),
            pltpu.VMEM((K * _CC, D), jnp.float32),
            pltpu.VMEM((_CC, D), jnp.float32),
            pltpu.SemaphoreType.DMA,
        ],
    )
    def k(hw_hbm, pos_hbm, out_hbm, idx_v, rows_v, out_v, sem):
        wid = lax.axis_index("s") * NC + lax.axis_index("c")
        tbase = wid * _TW
        pltpu.sync_copy(pos_hbm.at[pl.ds(K * tbase, K * _TW)], idx_v)
        for c in range(_TW // _CC):
            pltpu.async_copy(
                hw_hbm.at[idx_v.at[pl.ds(c * K * _CC, K * _CC)]], rows_v, sem
            ).wait()

            def body(t, carry):
                for dch in range(D // 16):
                    sl = pl.ds(dch * 16, 16)
                    out_v[t, sl] = rows_v[2 * t, sl] + rows_v[2 * t + 1, sl]
                return carry

            lax.fori_loop(0, _CC, body, 0)
            pltpu.sync_copy(out_v, out_hbm.at[pl.ds(tbase + c * _CC, _CC)])

    return k(hw, pos)


def kernel(x, Wg, bg, Wn, bn, W1, b1, W2, b2):
    x2 = x[0]
    noise = jax.random.normal(jax.random.PRNGKey(42), (1, T, E),
                              dtype=jnp.float32)
    h_logits = (x @ Wg + bg + noise * jax.nn.softplus(x @ Wn + bn))[0]
    w1, w2, a1, a2 = _gate(h_logits)
    st, sw, pos, meta = _route(a1, a2, w1, w2)
    xs = _sc_gather(x2, st)
    hw = _ffn(meta, xs, W1, b1, W2, b2, sw)
    out2 = _sc_combine(hw, pos)
    return out2[None, :, :]


# speedup vs baseline: 1.7459x; 1.0007x over previous
import functools

import jax
import jax.numpy as jnp
from jax import lax
from jax.experimental import pallas as pl
from jax.experimental.pallas import tpu as pltpu
from jax.experimental.pallas import tpu_sc as plsc

T, D = 2048, 768
E, K = 8, 2
FF = 4 * D

BT = 256
G = 24
R = G * BT

NC, NS = 2, 16
NW = NC * NS


def _gate_body(h_ref, w1_ref, w2_ref, a1_ref, a2_ref):
    h = h_ref[...]
    iota = lax.broadcasted_iota(jnp.int32, (T, E), 1)
    m1 = jnp.max(h, axis=1)
    a1 = jnp.min(jnp.where(h == m1[:, None], iota, E), axis=1)
    hm = jnp.where(iota == a1[:, None], -jnp.inf, h)
    m2 = jnp.max(hm, axis=1)
    a2 = jnp.min(jnp.where(hm == m2[:, None], iota, E), axis=1)
    d = jnp.exp(m2 - m1)
    w1_ref[...] = 1.0 / (1.0 + d)
    w2_ref[...] = d / (1.0 + d)
    a1_ref[...] = a1
    a2_ref[...] = a2


def _gate(h):
    return pl.pallas_call(
        _gate_body,
        out_shape=[
            jax.ShapeDtypeStruct((T,), jnp.float32),
            jax.ShapeDtypeStruct((T,), jnp.float32),
            jax.ShapeDtypeStruct((T,), jnp.int32),
            jax.ShapeDtypeStruct((T,), jnp.int32),
        ],
    )(h)


def _route(a1, a2, w1, w2):
    idx_flat = jnp.stack([a1, a2], axis=1).reshape(-1)
    w_flat = jnp.stack([w1, w2], axis=1).reshape(-1)
    onehot = (idx_flat[:, None] == jnp.arange(E)[None, :]).astype(jnp.int32)
    cs = jnp.cumsum(onehot, axis=0)
    counts = cs[-1]
    rank = jnp.sum((cs - onehot) * onehot, axis=1)
    alig = ((counts + BT - 1) // BT) * BT
    cum = jnp.cumsum(alig)
    offs = cum - alig
    pos = offs[idx_flat] + rank
    used = cum[-1] // BT
    tok = jnp.arange(T * K, dtype=jnp.int32) // K
    st = jnp.zeros((R,), jnp.int32).at[pos].set(tok)
    sw = jnp.zeros((R,), jnp.float32).at[pos].set(w_flat)
    bidx = jnp.arange(G, dtype=jnp.int32) * BT
    be_raw = jnp.searchsorted(cum, bidx, side="right").astype(jnp.int32)
    validb = bidx < cum[-1]
    last_e = jnp.clip(
        jnp.searchsorted(cum, cum[-1] - 1, side="right"), 0, E - 1
    ).astype(jnp.int32)
    be = jnp.where(validb, jnp.clip(be_raw, 0, E - 1), last_e)
    rb = jnp.where(validb, jnp.arange(G, dtype=jnp.int32),
                   jnp.maximum(used - 1, 0).astype(jnp.int32))
    meta = jnp.concatenate([be, rb, used[None].astype(jnp.int32)])
    return st, sw, pos.astype(jnp.int32), meta


_RW = R // NW
_GC = 64
_NGC = _RW // _GC


def _sc_gather(x2, st3):
    mesh = plsc.VectorSubcoreMesh(core_axis_name="c", subcore_axis_name="s")

    @functools.partial(
        pl.kernel, mesh=mesh,
        out_type=jax.ShapeDtypeStruct((R, D), jnp.float32),
        scratch_types=[
            pltpu.VMEM((_NGC, _GC), jnp.int32),
            pltpu.VMEM((_GC, D), jnp.float32),
            pltpu.VMEM((_GC, D), jnp.float32),
            pltpu.SemaphoreType.DMA,
            pltpu.SemaphoreType.DMA,
        ],
    )
    def k(x_hbm, st_hbm, out_hbm, idx_v, buf0, buf1, sem0, sem1):
        wid = lax.axis_index("s") * NC + lax.axis_index("c")
        base = wid * _RW
        pltpu.sync_copy(st_hbm.at[wid], idx_v)
        bufs, sems = (buf0, buf1), (sem0, sem1)
        hs = [pltpu.async_copy(x_hbm.at[idx_v.at[c]], bufs[c % 2], sems[c % 2])
              for c in range(2)]
        for c in range(_NGC):
            hs[c % 2].wait()
            pltpu.sync_copy(bufs[c % 2], out_hbm.at[pl.ds(base + c * _GC, _GC)])
            if c + 2 < _NGC:
                hs[c % 2] = pltpu.async_copy(
                    x_hbm.at[idx_v.at[c + 2]], bufs[c % 2], sems[c % 2])

    return k(x2, st3)


def _ffn_body(m_ref, xs_ref, W1_ref, b1_ref, W2_ref, b2_ref, w_ref, out_ref):
    b = pl.program_id(0)

    @pl.when(b < m_ref[2 * G])
    def _():
        h1 = lax.dot_general(
            xs_ref[...].astype(jnp.bfloat16), W1_ref[0].astype(jnp.bfloat16),
            (((1,), (0,)), ((), ())),
            preferred_element_type=jnp.float32)
        h1 = jnp.maximum(h1 + b1_ref[0, 0][None, :], 0.0)
        y = lax.dot_general(
            h1.astype(jnp.bfloat16), W2_ref[0].astype(jnp.bfloat16),
            (((1,), (0,)), ((), ())),
            preferred_element_type=jnp.float32)
        y = y + b2_ref[0, 0][None, :]
        out_ref[...] = y * w_ref[...][:, None]


def _ffn(meta, xs, W1, b1, W2, b2, sw):
    grid_spec = pltpu.PrefetchScalarGridSpec(
        num_scalar_prefetch=1,
        grid=(G,),
        in_specs=[
            pl.BlockSpec((BT, D), lambda b, m: (m[G + b], 0)),
            pl.BlockSpec((1, D, FF), lambda b, m: (m[b], 0, 0)),
            pl.BlockSpec((1, 1, FF), lambda b, m: (m[b], 0, 0)),
            pl.BlockSpec((1, FF, D), lambda b, m: (m[b], 0, 0)),
            pl.BlockSpec((1, 1, D), lambda b, m: (m[b], 0, 0)),
            pl.BlockSpec((BT,), lambda b, m: (m[G + b],)),
        ],
        out_specs=pl.BlockSpec((BT, D), lambda b, m: (m[G + b], 0)),
    )
    return pl.pallas_call(
        _ffn_body,
        grid_spec=grid_spec,
        out_shape=jax.ShapeDtypeStruct((R, D), jnp.float32),
        compiler_params=pltpu.CompilerParams(
            dimension_semantics=("arbitrary",)),
    )(meta, xs, W1, b1.reshape(E, 1, FF), W2, b2.reshape(E, 1, D), sw)


_TW = T // NW
_CC = 32


def _sc_combine(hw, pos):
    mesh = plsc.VectorSubcoreMesh(core_axis_name="c", subcore_axis_name="s")

    @functools.partial(
        pl.kernel, mesh=mesh,
        out_type=jax.ShapeDtypeStruct((T, D), jnp.float32),
        scratch_types=[
            pltpu.VMEM((_TW // _CC, K * _CC), jnp.int32),
            pltpu.VMEM((K * _CC, D), jnp.float32),
            pltpu.VMEM((K * _CC, D), jnp.float32),
            pltpu.VMEM((_CC, D), jnp.float32),
            pltpu.SemaphoreType.DMA,
            pltpu.SemaphoreType.DMA,
        ],
    )
    def k(hw_hbm, pos_hbm, out_hbm, idx_v, rows0, rows1, out_v, sem0, sem1):
        wid = lax.axis_index("s") * NC + lax.axis_index("c")
        tbase = wid * _TW
        pltpu.sync_copy(pos_hbm.at[wid], idx_v)
        bufs, sems = (rows0, rows1), (sem0, sem1)
        hs = [pltpu.async_copy(hw_hbm.at[idx_v.at[c]], bufs[c], sems[c])
              for c in range(2)]
        for c in range(_TW // _CC):
            hs[c].wait()
            rows_v = bufs[c]

            def body(t, carry):
                for dch in range(D // 16):
                    sl = pl.ds(dch * 16, 16)
                    out_v[t, sl] = rows_v[2 * t, sl] + rows_v[2 * t + 1, sl]
                return carry

            lax.fori_loop(0, _CC, body, 0)
            pltpu.sync_copy(out_v, out_hbm.at[pl.ds(tbase + c * _CC, _CC)])

    return k(hw, pos)


def kernel(x, Wg, bg, Wn, bn, W1, b1, W2, b2):
    x2 = x[0]
    noise = jax.random.normal(jax.random.PRNGKey(42), (1, T, E),
                              dtype=jnp.float32)
    h_logits = (x @ Wg + bg + noise * jax.nn.softplus(x @ Wn + bn))[0]
    w1, w2, a1, a2 = _gate(h_logits)
    st, sw, pos, meta = _route(a1, a2, w1, w2)
    xs = _sc_gather(x2, st.reshape(NW, _NGC, _GC))
    hw = _ffn(meta, xs, W1, b1, W2, b2, sw)
    out2 = _sc_combine(hw, pos.reshape(NW, _TW // _CC, K * _CC))
    return out2[None, :, :]


# speedup vs baseline: 2.5106x; 1.4380x over previous
import functools

import jax
import jax.numpy as jnp
from jax import lax
from jax.experimental import pallas as pl
from jax.experimental.pallas import tpu as pltpu
from jax.experimental.pallas import tpu_sc as plsc

T, D = 2048, 768
E, K = 8, 2
FF = 4 * D

BT = 256
G = 24
R = G * BT

NC, NS = 2, 16
NW = NC * NS


def _gate_body(h_ref, w1_ref, w2_ref, a1_ref, a2_ref):
    h = h_ref[...]
    iota = lax.broadcasted_iota(jnp.int32, (T, E), 1)
    m1 = jnp.max(h, axis=1)
    a1 = jnp.min(jnp.where(h == m1[:, None], iota, E), axis=1)
    hm = jnp.where(iota == a1[:, None], -jnp.inf, h)
    m2 = jnp.max(hm, axis=1)
    a2 = jnp.min(jnp.where(hm == m2[:, None], iota, E), axis=1)
    d = jnp.exp(m2 - m1)
    w1_ref[...] = 1.0 / (1.0 + d)
    w2_ref[...] = d / (1.0 + d)
    a1_ref[...] = a1
    a2_ref[...] = a2


def _gate(h):
    return pl.pallas_call(
        _gate_body,
        out_shape=[
            jax.ShapeDtypeStruct((T,), jnp.float32),
            jax.ShapeDtypeStruct((T,), jnp.float32),
            jax.ShapeDtypeStruct((T,), jnp.int32),
            jax.ShapeDtypeStruct((T,), jnp.int32),
        ],
    )(h)


def _route(a1, a2):
    idx_flat = jnp.stack([a1, a2], axis=1).reshape(-1)
    onehot = (idx_flat[:, None] == jnp.arange(E)[None, :]).astype(jnp.int32)
    cs = jnp.cumsum(onehot, axis=0)
    counts = cs[-1]
    rank = jnp.sum((cs - onehot) * onehot, axis=1)
    alig = ((counts + BT - 1) // BT) * BT
    cum = jnp.cumsum(alig)
    offs = cum - alig
    pos = (offs[idx_flat] + rank).astype(jnp.int32)
    used = cum[-1] // BT
    bidx = jnp.arange(G, dtype=jnp.int32) * BT
    be_raw = jnp.searchsorted(cum, bidx, side="right").astype(jnp.int32)
    be = jnp.clip(be_raw, 0, E - 1)
    meta = jnp.concatenate([be, used[None].astype(jnp.int32)])
    return pos, meta


def _ffn_body(m_ref, xb_ref, p0_ref, p1_ref, w0_ref, w1_ref,
              W1_ref, b1_ref, W2_ref, b2_ref, out_ref):
    b = pl.program_id(0)

    @pl.when(b < m_ref[G])
    def _():
        slot = lax.broadcasted_iota(jnp.int32, (BT, T), 0) + b * BT
        eq0 = slot == p0_ref[...][None, :]
        eq1 = slot == p1_ref[...][None, :]
        sel = (eq0 | eq1).astype(jnp.bfloat16)
        xs = lax.dot_general(
            sel, xb_ref[...], (((1,), (0,)), ((), ())),
            preferred_element_type=jnp.float32).astype(jnp.bfloat16)
        wslot = jnp.sum(
            jnp.where(eq0, w0_ref[...][None, :], 0.0)
            + jnp.where(eq1, w1_ref[...][None, :], 0.0), axis=1)
        h1 = lax.dot_general(
            xs, W1_ref[0].astype(jnp.bfloat16), (((1,), (0,)), ((), ())),
            preferred_element_type=jnp.float32)
        h1 = jnp.maximum(h1 + b1_ref[0, 0][None, :], 0.0)
        y = lax.dot_general(
            h1.astype(jnp.bfloat16), W2_ref[0].astype(jnp.bfloat16),
            (((1,), (0,)), ((), ())),
            preferred_element_type=jnp.float32)
        y = y + b2_ref[0, 0][None, :]
        out_ref[...] = y * wslot[:, None]


def _ffn(meta, xb, pos, w1, w2, W1, b1, W2, b2):
    pos2 = pos.reshape(T, K)
    grid_spec = pltpu.PrefetchScalarGridSpec(
        num_scalar_prefetch=1,
        grid=(G,),
        in_specs=[
            pl.BlockSpec((T, D), lambda b, m: (0, 0)),
            pl.BlockSpec((T,), lambda b, m: (0,)),
            pl.BlockSpec((T,), lambda b, m: (0,)),
            pl.BlockSpec((T,), lambda b, m: (0,)),
            pl.BlockSpec((T,), lambda b, m: (0,)),
            pl.BlockSpec((1, D, FF), lambda b, m: (m[b], 0, 0)),
            pl.BlockSpec((1, 1, FF), lambda b, m: (m[b], 0, 0)),
            pl.BlockSpec((1, FF, D), lambda b, m: (m[b], 0, 0)),
            pl.BlockSpec((1, 1, D), lambda b, m: (m[b], 0, 0)),
        ],
        out_specs=pl.BlockSpec((BT, D), lambda b, m: (b, 0)),
    )
    return pl.pallas_call(
        _ffn_body,
        grid_spec=grid_spec,
        out_shape=jax.ShapeDtypeStruct((R, D), jnp.float32),
        compiler_params=pltpu.CompilerParams(
            dimension_semantics=("arbitrary",)),
    )(meta, xb, pos2[:, 0], pos2[:, 1], w1, w2,
      W1, b1.reshape(E, 1, FF), W2, b2.reshape(E, 1, D))


_TW = T // NW
_CC = 32


def _sc_combine(hw, pos):
    mesh = plsc.VectorSubcoreMesh(core_axis_name="c", subcore_axis_name="s")

    @functools.partial(
        pl.kernel, mesh=mesh,
        out_type=jax.ShapeDtypeStruct((T, D), jnp.float32),
        scratch_types=[
            pltpu.VMEM((_TW // _CC, K * _CC), jnp.int32),
            pltpu.VMEM((K * _CC, D), jnp.float32),
            pltpu.VMEM((K * _CC, D), jnp.float32),
            pltpu.VMEM((_CC, D), jnp.float32),
            pltpu.SemaphoreType.DMA,
            pltpu.SemaphoreType.DMA,
        ],
    )
    def k(hw_hbm, pos_hbm, out_hbm, idx_v, rows0, rows1, out_v, sem0, sem1):
        wid = lax.axis_index("s") * NC + lax.axis_index("c")
        tbase = wid * _TW
        pltpu.sync_copy(pos_hbm.at[wid], idx_v)
        bufs, sems = (rows0, rows1), (sem0, sem1)
        hs = [pltpu.async_copy(hw_hbm.at[idx_v.at[c]], bufs[c], sems[c])
              for c in range(2)]
        for c in range(_TW // _CC):
            hs[c].wait()
            rows_v = bufs[c]

            def body(t, carry):
                for dch in range(D // 16):
                    sl = pl.ds(dch * 16, 16)
                    out_v[t, sl] = rows_v[2 * t, sl] + rows_v[2 * t + 1, sl]
                return carry

            lax.fori_loop(0, _CC, body, 0)
            pltpu.sync_copy(out_v, out_hbm.at[pl.ds(tbase + c * _CC, _CC)])

    return k(hw, pos)


def kernel(x, Wg, bg, Wn, bn, W1, b1, W2, b2):
    x2 = x[0]
    noise = jax.random.normal(jax.random.PRNGKey(42), (1, T, E),
                              dtype=jnp.float32)
    h_logits = (x @ Wg + bg + noise * jax.nn.softplus(x @ Wn + bn))[0]
    w1, w2, a1, a2 = _gate(h_logits)
    pos, meta = _route(a1, a2)
    hw = _ffn(meta, x2.astype(jnp.bfloat16), pos, w1, w2, W1, b1, W2, b2)
    out2 = _sc_combine(hw, pos.reshape(NW, _TW // _CC, K * _CC))
    return out2[None, :, :]
